# Initial kernel scaffold; baseline (speedup 1.0000x reference)
import jax
import jax.numpy as jnp
from jax.experimental import pallas as pl


def kernel(filtered_sino, rows, cols, vals):
    raise NotImplementedError("write your pallas kernel here")


# speedup vs baseline: 4.3157x; 4.3157x over previous
import functools

import jax
import jax.numpy as jnp
from jax import lax
from jax.experimental import pallas as pl
from jax.experimental.pallas import tpu as pltpu
from jax.experimental.pallas import tpu_sc as plsc

PROJ = 1000
DET = 513
IMG = 362
BSZ = 5
NNZ = 4_000_000
NROWS = IMG * IMG
NCOLS = DET * PROJ
NROWS_PAD = 131072
NCOLS_PAD = 513024
NJOBS = BSZ + 1

L = 128
IDXROWS = NNZ // L
PER_SC = IDXROWS // 2
BASE_CNT = PER_SC // 16
EXTRA = PER_SC - 16 * BASE_CNT

TBL_SLC = NCOLS_PAD // 16
ACC_SLC = NROWS_PAD // 16

_mesh = plsc.VectorSubcoreMesh(core_axis_name="c", subcore_axis_name="s")


@functools.partial(
    pl.kernel,
    out_type=jax.ShapeDtypeStruct((2 * NJOBS * NROWS_PAD,), jnp.float32),
    mesh=_mesh,
    scratch_types=[
        pltpu.VMEM_SHARED((NCOLS_PAD,), jnp.float32),
        pltpu.VMEM_SHARED((NROWS_PAD,), jnp.float32),
        pltpu.VMEM((L,), jnp.int32),
        pltpu.VMEM((L,), jnp.int32),
        pltpu.VMEM((L,), jnp.float32),
        pltpu.VMEM((L,), jnp.float32),
        pltpu.VMEM((L,), jnp.float32),
        pltpu.VMEM((TBL_SLC,), jnp.float32),
        pltpu.SemaphoreType.DMA,
    ],
)
def _bp_sc(flat6, rows2, cols2, vals2, zeros_hbm, out,
           table, acc, rowbuf, colbuf, valbuf, gathbuf, prodbuf, stage, sem):
    c = lax.axis_index("c")
    s = lax.axis_index("s")
    base = c * PER_SC + s * BASE_CNT + jnp.minimum(s, EXTRA)
    count = BASE_CNT + (s < EXTRA).astype(jnp.int32)

    def job_body(j, carry):
        pltpu.sync_copy(zeros_hbm, stage.at[pl.ds(0, ACC_SLC)])
        pltpu.sync_copy(stage.at[pl.ds(0, ACC_SLC)],
                        acc.at[pl.ds(s * ACC_SLC, ACC_SLC)])
        tbl_off = pl.multiple_of(j * NCOLS_PAD + s * TBL_SLC, 8)
        pltpu.sync_copy(flat6.at[pl.ds(tbl_off, TBL_SLC)], stage)
        pltpu.sync_copy(stage, table.at[pl.ds(s * TBL_SLC, TBL_SLC)])
        plsc.subcore_barrier()

        def row_body(r, carry2):
            off = pl.multiple_of(r * L, 8)
            pltpu.sync_copy(cols2.at[pl.ds(off, L)], colbuf)
            pltpu.sync_copy(vals2.at[pl.ds(off, L)], valbuf)
            pltpu.sync_copy(rows2.at[pl.ds(off, L)], rowbuf)
            pltpu.async_copy(table.at[colbuf], gathbuf, sem).wait()
            for v in range(L // 16):
                sl = pl.ds(v * 16, 16)
                prodbuf[sl] = valbuf[sl] * gathbuf[sl]
            pltpu.async_copy(prodbuf, acc.at[rowbuf], sem, add=True).wait()
            return carry2

        lax.fori_loop(base, base + count, row_body, 0)
        plsc.subcore_barrier()
        out_off = pl.multiple_of((c * NJOBS + j) * NROWS_PAD + s * ACC_SLC, 8)
        pltpu.sync_copy(acc.at[pl.ds(s * ACC_SLC, ACC_SLC)],
                        stage.at[pl.ds(0, ACC_SLC)])
        pltpu.sync_copy(stage.at[pl.ds(0, ACC_SLC)], out.at[pl.ds(out_off, ACC_SLC)])
        return carry

    lax.fori_loop(0, NJOBS, job_body, 0)


def kernel(filtered_sino, rows, cols, vals):
    flat = filtered_sino.reshape(BSZ, NCOLS)
    flat6 = jnp.concatenate([flat, jnp.ones((1, NCOLS), jnp.float32)], axis=0)
    flat6 = jnp.pad(flat6, ((0, 0), (0, NCOLS_PAD - NCOLS)))
    zeros = jnp.zeros((ACC_SLC,), jnp.float32)

    out = _bp_sc(flat6.reshape(-1), rows, cols, vals, zeros)
    out = out.reshape(2, NJOBS, NROWS_PAD)
    tot = out[0] + out[1]
    backward = tot[:BSZ, :NROWS]
    sens = tot[BSZ, :NROWS]
    return (backward / sens[None, :]).reshape(BSZ, IMG, IMG, 1)


# speedup vs baseline: 40.9991x; 9.5000x over previous
import functools

import jax
import jax.numpy as jnp
from jax import lax
from jax.experimental import pallas as pl
from jax.experimental.pallas import tpu as pltpu
from jax.experimental.pallas import tpu_sc as plsc

PROJ = 1000
DET = 513
IMG = 362
BSZ = 5
NNZ = 4_000_000
NROWS = IMG * IMG
NCOLS = DET * PROJ
NROWS_PAD = 131072
NCOLS_PAD = 513024

L = 128
IDXROWS = NNZ // L
CH = 61
CHN = CH * L
CHUNKS = 512
TAILROWS = IDXROWS - CHUNKS * CH
PER_SC_CH = CHUNKS // 2
PER_TILE_CH = PER_SC_CH // 16
PAIRS = PER_TILE_CH // 2

TBL_SLC = 32064
TBL_LAST = NCOLS - 15 * TBL_SLC
TP = 4008
TP_LAST = TBL_LAST - 7 * TP
ACC_SLC = NROWS_PAD // 16
ZB = 4096

_mesh = plsc.VectorSubcoreMesh(core_axis_name="c", subcore_axis_name="s")


@functools.partial(
    pl.kernel,
    out_type=jax.ShapeDtypeStruct((2 * (BSZ + 1) * NROWS_PAD,), jnp.float32),
    mesh=_mesh,
    scratch_types=[
        pltpu.VMEM_SHARED((NCOLS_PAD,), jnp.float32),
        pltpu.VMEM_SHARED((NROWS_PAD,), jnp.float32),
        pltpu.VMEM_SHARED((NROWS_PAD,), jnp.float32),
        pltpu.VMEM((CHN,), jnp.int32),
        pltpu.VMEM((CHN,), jnp.int32),
        pltpu.VMEM((CHN,), jnp.int32),
        pltpu.VMEM((CHN,), jnp.int32),
        pltpu.VMEM((CHN,), jnp.float32),
        pltpu.VMEM((CHN,), jnp.float32),
        pltpu.VMEM((CHN,), jnp.float32),
        pltpu.VMEM((CHN,), jnp.float32),
        pltpu.VMEM((CHN,), jnp.float32),
        pltpu.VMEM((ZB,), jnp.float32),
        pltpu.VMEM((L,), jnp.int32),
        pltpu.VMEM((L,), jnp.int32),
        pltpu.VMEM((L,), jnp.float32),
        pltpu.VMEM((L,), jnp.float32),
        pltpu.VMEM((L,), jnp.float32),
        pltpu.SemaphoreType.DMA,
        pltpu.SemaphoreType.DMA,
        pltpu.SemaphoreType.DMA,
        pltpu.SemaphoreType.DMA,
        pltpu.SemaphoreType.DMA,
        pltpu.SemaphoreType.DMA,
        pltpu.SemaphoreType.DMA,
        pltpu.SemaphoreType.DMA,
    ],
)
def _bp_sc(flat, rows, cols, vals, zeros_hbm, out,
           table, acc, acc2,
           rbuf0, rbuf1, cbuf0, cbuf1, vbuf0, vbuf1, gbuf, pbuf0, pbuf1,
           zbuf, trowb, tcolb, tvalb, tgathb, tprodb,
           sem_st0, sem_st1, sem_g, sem_sc0, sem_sc1, sem_s20, sem_s21, sem_t):
    c = lax.axis_index("c")
    s = lax.axis_index("s")
    rbuf = (rbuf0, rbuf1)
    cbuf = (cbuf0, cbuf1)
    vbuf = (vbuf0, vbuf1)
    pbuf = (pbuf0, pbuf1)
    sem_st = (sem_st0, sem_st1)
    sem_sc = (sem_sc0, sem_sc1)
    sem_s2 = (sem_s20, sem_s21)

    my_chunk0 = c * PER_SC_CH + s * PER_TILE_CH

    pltpu.sync_copy(zeros_hbm, zbuf)

    def nnz_off(k):
        return pl.multiple_of((my_chunk0 + k) * CHN, 8)

    def fire_stage(b, k):
        off = nnz_off(k)
        pltpu.async_copy(rows.at[pl.ds(off, CHN)], rbuf[b], sem_st[b])
        pltpu.async_copy(cols.at[pl.ds(off, CHN)], cbuf[b], sem_st[b])
        pltpu.async_copy(vals.at[pl.ds(off, CHN)], vbuf[b], sem_st[b])

    def wait_stage(b, k):
        off = nnz_off(k)
        pltpu.make_async_copy(rows.at[pl.ds(off, CHN)], rbuf[b], sem_st[b]).wait()
        pltpu.make_async_copy(cols.at[pl.ds(off, CHN)], cbuf[b], sem_st[b]).wait()
        pltpu.make_async_copy(vals.at[pl.ds(off, CHN)], vbuf[b], sem_st[b]).wait()

    def wait_scatter(b, with_sens):
        pltpu.make_async_copy(pbuf[b], acc.at[rbuf[b]], sem_sc[b]).wait()

        @pl.when(with_sens)
        def _():
            pltpu.make_async_copy(vbuf[b], acc2.at[rbuf[b]], sem_s2[b]).wait()

    def job_body(j, carry):
        with_sens = j == 0

        for h in range(ACC_SLC // ZB):
            pltpu.sync_copy(zbuf, acc.at[pl.ds(s * ACC_SLC + h * ZB, ZB)])

        @pl.when(with_sens)
        def _():
            for h in range(ACC_SLC // ZB):
                pltpu.sync_copy(zbuf, acc2.at[pl.ds(s * ACC_SLC + h * ZB, ZB)])

        tbl_off = pl.multiple_of(j * NCOLS + s * TBL_SLC, 8)

        @pl.when(s < 15)
        def _():
            for h in range(TBL_SLC // TP):
                src = flat.at[pl.ds(pl.multiple_of(tbl_off + h * TP, 8), TP)]
                pltpu.sync_copy(src, pbuf0.at[pl.ds(0, TP)])
                pltpu.sync_copy(pbuf0.at[pl.ds(0, TP)],
                                table.at[pl.ds(s * TBL_SLC + h * TP, TP)])

        @pl.when(s == 15)
        def _():
            for h in range(7):
                src = flat.at[pl.ds(pl.multiple_of(tbl_off + h * TP, 8), TP)]
                pltpu.sync_copy(src, pbuf0.at[pl.ds(0, TP)])
                pltpu.sync_copy(pbuf0.at[pl.ds(0, TP)],
                                table.at[pl.ds(15 * TBL_SLC + h * TP, TP)])
            src = flat.at[pl.ds(pl.multiple_of(tbl_off + 7 * TP, 8), TP_LAST)]
            pltpu.sync_copy(src, pbuf0.at[pl.ds(0, TP_LAST)])
            pltpu.sync_copy(pbuf0.at[pl.ds(0, TP_LAST)],
                            table.at[pl.ds(15 * TBL_SLC + 7 * TP, TP_LAST)])

        plsc.subcore_barrier()

        fire_stage(0, 0)

        def chunk_body(b, k):
            nb = 1 - b
            wait_stage(b, k)
            gd = pltpu.async_copy(table.at[cbuf[b]], gbuf, sem_g)

            @pl.when(k > 0)
            def _():
                wait_scatter(nb, with_sens)

            @pl.when(k < PER_TILE_CH - 1)
            def _():
                fire_stage(nb, k + 1)

            gd.wait()

            def mul_body(q, carry2):
                o = pl.multiple_of(q * 64, 16)
                for u in range(4):
                    sl = pl.ds(o + u * 16, 16)
                    pbuf[b][sl] = vbuf[b][sl] * gbuf[sl]
                return carry2

            lax.fori_loop(0, CHN // 64, mul_body, 0)

            pltpu.async_copy(pbuf[b], acc.at[rbuf[b]], sem_sc[b], add=True)

            @pl.when(with_sens)
            def _():
                pltpu.async_copy(vbuf[b], acc2.at[rbuf[b]], sem_s2[b], add=True)

        def pair_body(p, carry2):
            chunk_body(0, 2 * p)
            chunk_body(1, 2 * p + 1)
            return carry2

        lax.fori_loop(0, PAIRS, pair_body, 0)
        wait_scatter(1, with_sens)

        @pl.when(s == 15)
        def _():
            trow0 = CHUNKS * CH + (TAILROWS // 2) * c

            def tail_body(t, carry2):
                off = pl.multiple_of((trow0 + t) * L, 8)
                pltpu.sync_copy(rows.at[pl.ds(off, L)], trowb)
                pltpu.sync_copy(cols.at[pl.ds(off, L)], tcolb)
                pltpu.sync_copy(vals.at[pl.ds(off, L)], tvalb)
                pltpu.async_copy(table.at[tcolb], tgathb, sem_t).wait()
                for u in range(L // 16):
                    sl = pl.ds(u * 16, 16)
                    tprodb[sl] = tvalb[sl] * tgathb[sl]
                pltpu.async_copy(tprodb, acc.at[trowb], sem_t, add=True).wait()

                @pl.when(with_sens)
                def _():
                    pltpu.async_copy(tvalb, acc2.at[trowb], sem_t,
                                     add=True).wait()
                return carry2

            lax.fori_loop(0, TAILROWS // 2, tail_body, 0)

        plsc.subcore_barrier()

        out_off = pl.multiple_of((c * (BSZ + 1) + j) * NROWS_PAD + s * ACC_SLC, 8)
        for h in range(ACC_SLC // ZB):
            pltpu.sync_copy(acc.at[pl.ds(s * ACC_SLC + h * ZB, ZB)], zbuf)
            pltpu.sync_copy(zbuf, out.at[pl.ds(
                pl.multiple_of(out_off + h * ZB, 8), ZB)])

        @pl.when(with_sens)
        def _():
            sens_off = pl.multiple_of(
                (c * (BSZ + 1) + BSZ) * NROWS_PAD + s * ACC_SLC, 8)
            for h in range(ACC_SLC // ZB):
                pltpu.sync_copy(acc2.at[pl.ds(s * ACC_SLC + h * ZB, ZB)], zbuf)
                pltpu.sync_copy(zbuf, out.at[pl.ds(
                    pl.multiple_of(sens_off + h * ZB, 8), ZB)])

        pltpu.sync_copy(zeros_hbm, zbuf)
        return carry

    lax.fori_loop(0, BSZ, job_body, 0)


def kernel(filtered_sino, rows, cols, vals):
    flat = filtered_sino.reshape(-1)
    zeros = jnp.zeros((ZB,), jnp.float32)

    out = _bp_sc(flat, rows, cols, vals, zeros)
    out = out.reshape(2, BSZ + 1, NROWS_PAD)
    tot = out[0] + out[1]
    backward = tot[:BSZ, :NROWS]
    sens = tot[BSZ, :NROWS]
    return (backward / sens[None, :]).reshape(BSZ, IMG, IMG, 1)
